# Initial kernel scaffold; baseline (speedup 1.0000x reference)
import jax
import jax.numpy as jnp
from jax.experimental import pallas as pl


def kernel(x, edge_index, W_msg1, W_self1, b1, W_msg2, W_self2, b2):
    raise NotImplementedError("write your pallas kernel here")


# speedup vs baseline: 4.6342x; 4.6342x over previous
import functools

import jax
import jax.numpy as jnp
from jax import lax
from jax.experimental import pallas as pl
from jax.experimental.pallas import tpu as pltpu
from jax.experimental.pallas import tpu_sc as plsc

N = 10000
E = 320000
D = 128
NC = 2
NS = 16
NW = NC * NS
E_PER_W = E // NW
CHUNK = 80
N_CHUNKS = E_PER_W // CHUNK
ROWS_PER_TILE = 624
ROWS_TAIL_START = NS * ROWS_PER_TILE
ROWS_TAIL = N - ROWS_TAIL_START

_MESH = plsc.VectorSubcoreMesh(core_axis_name="c", subcore_axis_name="s")


def _sc_segment_sum(h, src, dst, zrows, iota, gather):
    @functools.partial(
        pl.kernel,
        out_type=jax.ShapeDtypeStruct((NC, N, D), jnp.float32),
        mesh=_MESH,
        scratch_types=[
            pltpu.VMEM_SHARED((N, D), jnp.float32),
            pltpu.VMEM((CHUNK,), jnp.int32),
            pltpu.VMEM((CHUNK,), jnp.int32),
            pltpu.VMEM((CHUNK, D), jnp.float32),
            pltpu.VMEM((CHUNK,), jnp.int32),
            pltpu.VMEM((64,), jnp.int32),
            pltpu.VMEM((ROWS_TAIL,), jnp.int32),
            pltpu.SemaphoreType.DMA,
        ],
    )
    def k(h_hbm, src_hbm, dst_hbm, zrows_hbm, iota_hbm,
          acc_out, s_acc, src_v, dst_v, rows_v, ii80, ii64, ii16, sem):
        cid = lax.axis_index("c")
        sid = lax.axis_index("s")
        base = (cid * NS + sid) * E_PER_W
        row0 = sid * ROWS_PER_TILE

        def slabs(fn):
            for j in range(7):
                fn(ii80, row0 + j * CHUNK, CHUNK)
            fn(ii64, row0 + 7 * CHUNK, 64)

        pltpu.sync_copy(zrows_hbm, rows_v)

        def init(ii, off, sz):
            pltpu.sync_copy(iota_hbm.at[pl.ds(off, sz)], ii)
            pltpu.sync_copy(rows_v.at[pl.ds(0, sz)], s_acc.at[ii])
        slabs(init)
        @pl.when(sid == 0)
        def _():
            init(ii16, ROWS_TAIL_START, ROWS_TAIL)
        if not gather:
            pltpu.sync_copy(h_hbm, rows_v)
        plsc.subcore_barrier()

        def body(i, carry):
            off = base + i * CHUNK
            pltpu.sync_copy(dst_hbm.at[pl.ds(off, CHUNK)], dst_v)
            if gather:
                pltpu.sync_copy(src_hbm.at[pl.ds(off, CHUNK)], src_v)
                pltpu.async_copy(h_hbm.at[src_v], rows_v, sem).wait()
            pltpu.sync_copy(rows_v, s_acc.at[dst_v], add=True)
            return carry

        lax.fori_loop(0, N_CHUNKS, body, 0)
        plsc.subcore_barrier()

        def readout(ii, off, sz):
            pltpu.sync_copy(iota_hbm.at[pl.ds(off, sz)], ii)
            pltpu.async_copy(s_acc.at[ii], rows_v.at[pl.ds(0, sz)], sem).wait()
            pltpu.sync_copy(rows_v.at[pl.ds(0, sz)],
                            acc_out.at[cid, pl.ds(off, sz)])
        slabs(readout)
        @pl.when(sid == 0)
        def _():
            readout(ii16, ROWS_TAIL_START, ROWS_TAIL)

    return k(h, src, dst, zrows, iota)


_BLK = 1000


def _tc_combine_mm(part, degp, h, w_msg, w_self, b, relu):
    def body(part_ref, deg_ref, h_ref, wm_ref, ws_ref, b_ref, o_ref):
        d = jnp.maximum(deg_ref[0] + deg_ref[1], 1.0)
        agg = (part_ref[0] + part_ref[1]) / d
        r = (jnp.dot(agg, wm_ref[...], preferred_element_type=jnp.float32)
             + jnp.dot(h_ref[...], ws_ref[...], preferred_element_type=jnp.float32)
             + b_ref[...])
        if relu:
            r = jnp.maximum(r, 0.0)
        o_ref[...] = r

    return pl.pallas_call(
        body,
        grid=(N // _BLK,),
        in_specs=[
            pl.BlockSpec((NC, _BLK, D), lambda i: (0, i, 0)),
            pl.BlockSpec((NC, _BLK, 1), lambda i: (0, i, 0)),
            pl.BlockSpec((_BLK, D), lambda i: (i, 0)),
            pl.BlockSpec((D, D), lambda i: (0, 0)),
            pl.BlockSpec((D, D), lambda i: (0, 0)),
            pl.BlockSpec((1, D), lambda i: (0, 0)),
        ],
        out_specs=pl.BlockSpec((_BLK, D), lambda i: (i, 0)),
        out_shape=jax.ShapeDtypeStruct((N, D), jnp.float32),
    )(part, degp, h, w_msg, w_self, b)


def kernel(x, edge_index, W_msg1, W_self1, b1, W_msg2, W_self2, b2):
    src = edge_index[0].astype(jnp.int32)
    dst = edge_index[1].astype(jnp.int32)
    iota = jnp.arange(N, dtype=jnp.int32)
    zrows = jnp.zeros((CHUNK, D), jnp.float32)
    ones = jnp.ones((CHUNK, D), jnp.float32)

    deg_parts = _sc_segment_sum(ones, src, dst, zrows, iota, gather=False)
    degp = deg_parts[:, :, 0:1]

    acc1 = _sc_segment_sum(x, src, dst, zrows, iota, gather=True)
    h = _tc_combine_mm(acc1, degp, x, W_msg1, W_self1, b1.reshape(1, D),
                       relu=True)
    acc2 = _sc_segment_sum(h, src, dst, zrows, iota, gather=True)
    out = _tc_combine_mm(acc2, degp, h, W_msg2, W_self2, b2.reshape(1, D),
                         relu=False)
    return out


# speedup vs baseline: 6.6497x; 1.4349x over previous
import functools

import jax
import jax.numpy as jnp
from jax import lax
from jax.experimental import pallas as pl
from jax.experimental.pallas import tpu as pltpu
from jax.experimental.pallas import tpu_sc as plsc

N = 10000
E = 320000
D = 128
NC = 2
NS = 16
NW = NC * NS
E_PER_W = E // NW
CHUNK = 80
N_CHUNKS = E_PER_W // CHUNK
ROWS_PER_TILE = 624
ROWS_TAIL_START = NS * ROWS_PER_TILE
ROWS_TAIL = N - ROWS_TAIL_START

_MESH = plsc.VectorSubcoreMesh(core_axis_name="c", subcore_axis_name="s")


def _sc_segment_sum(h, src, dst, zrows, iota, gather):
    @functools.partial(
        pl.kernel,
        out_type=jax.ShapeDtypeStruct((NC, N, D), jnp.float32),
        mesh=_MESH,
        scratch_types=[
            pltpu.VMEM_SHARED((N, D), jnp.float32),
            pltpu.VMEM((N_CHUNKS, CHUNK), jnp.int32),
            pltpu.VMEM((N_CHUNKS, CHUNK), jnp.int32),
            pltpu.VMEM((CHUNK, D), jnp.float32),
            pltpu.VMEM((CHUNK,), jnp.int32),
            pltpu.VMEM((64,), jnp.int32),
            pltpu.VMEM((ROWS_TAIL,), jnp.int32),
            pltpu.SemaphoreType.DMA,
        ],
    )
    def k(h_hbm, src_hbm, dst_hbm, zrows_hbm, iota_hbm,
          acc_out, s_acc, src_b, dst_b, rows_v, ii80, ii64, ii16, sem):
        cid = lax.axis_index("c")
        sid = lax.axis_index("s")
        wid = cid * NS + sid
        row0 = sid * ROWS_PER_TILE

        def slabs(fn):
            for j in range(7):
                fn(ii80, row0 + j * CHUNK, CHUNK)
            fn(ii64, row0 + 7 * CHUNK, 64)

        pltpu.sync_copy(zrows_hbm, rows_v)

        def init(ii, off, sz):
            pltpu.sync_copy(iota_hbm.at[pl.ds(off, sz)], ii)
            pltpu.sync_copy(rows_v.at[pl.ds(0, sz)], s_acc.at[ii])
        slabs(init)
        @pl.when(sid == 0)
        def _():
            init(ii16, ROWS_TAIL_START, ROWS_TAIL)
        pltpu.sync_copy(dst_hbm.at[wid], dst_b)
        if gather:
            pltpu.sync_copy(src_hbm.at[wid], src_b)
        else:
            pltpu.sync_copy(h_hbm, rows_v)
        plsc.subcore_barrier()

        def body(i, carry):
            if gather:
                pltpu.async_copy(h_hbm.at[src_b.at[i]], rows_v, sem).wait()
            pltpu.sync_copy(rows_v, s_acc.at[dst_b.at[i]], add=True)
            return carry

        lax.fori_loop(0, N_CHUNKS, body, 0)
        plsc.subcore_barrier()

        def readout(ii, off, sz):
            pltpu.sync_copy(iota_hbm.at[pl.ds(off, sz)], ii)
            pltpu.async_copy(s_acc.at[ii], rows_v.at[pl.ds(0, sz)], sem).wait()
            pltpu.sync_copy(rows_v.at[pl.ds(0, sz)],
                            acc_out.at[cid, pl.ds(off, sz)])
        slabs(readout)
        @pl.when(sid == 0)
        def _():
            readout(ii16, ROWS_TAIL_START, ROWS_TAIL)

    return k(h, src, dst, zrows, iota)


_BLK = 1000


def _tc_combine_mm(part, degp, h, w_msg, w_self, b, relu):
    def body(part_ref, deg_ref, h_ref, wm_ref, ws_ref, b_ref, o_ref):
        d = jnp.maximum(deg_ref[0] + deg_ref[1], 1.0)
        agg = (part_ref[0] + part_ref[1]) / d
        r = (jnp.dot(agg, wm_ref[...], preferred_element_type=jnp.float32)
             + jnp.dot(h_ref[...], ws_ref[...], preferred_element_type=jnp.float32)
             + b_ref[...])
        if relu:
            r = jnp.maximum(r, 0.0)
        o_ref[...] = r

    return pl.pallas_call(
        body,
        grid=(N // _BLK,),
        in_specs=[
            pl.BlockSpec((NC, _BLK, D), lambda i: (0, i, 0)),
            pl.BlockSpec((NC, _BLK, 1), lambda i: (0, i, 0)),
            pl.BlockSpec((_BLK, D), lambda i: (i, 0)),
            pl.BlockSpec((D, D), lambda i: (0, 0)),
            pl.BlockSpec((D, D), lambda i: (0, 0)),
            pl.BlockSpec((1, D), lambda i: (0, 0)),
        ],
        out_specs=pl.BlockSpec((_BLK, D), lambda i: (i, 0)),
        out_shape=jax.ShapeDtypeStruct((N, D), jnp.float32),
    )(part, degp, h, w_msg, w_self, b)


def kernel(x, edge_index, W_msg1, W_self1, b1, W_msg2, W_self2, b2):
    src = edge_index[0].astype(jnp.int32).reshape(NW, N_CHUNKS, CHUNK)
    dst = edge_index[1].astype(jnp.int32).reshape(NW, N_CHUNKS, CHUNK)
    iota = jnp.arange(N, dtype=jnp.int32)
    zrows = jnp.zeros((CHUNK, D), jnp.float32)
    ones = jnp.ones((CHUNK, D), jnp.float32)

    deg_parts = _sc_segment_sum(ones, src, dst, zrows, iota, gather=False)
    degp = deg_parts[:, :, 0:1]

    acc1 = _sc_segment_sum(x, src, dst, zrows, iota, gather=True)
    h = _tc_combine_mm(acc1, degp, x, W_msg1, W_self1, b1.reshape(1, D),
                       relu=True)
    acc2 = _sc_segment_sum(h, src, dst, zrows, iota, gather=True)
    out = _tc_combine_mm(acc2, degp, h, W_msg2, W_self2, b2.reshape(1, D),
                         relu=False)
    return out
